# baseline (speedup 1.0000x reference)
import functools

import jax
import jax.numpy as jnp
import numpy as np
from jax import lax
from jax.experimental import pallas as pl
from jax.experimental.pallas import tpu as pltpu
from jax.experimental.pallas import tpu_sc as plsc

B, T, C = 1, 2048, 1024
H, DH = 16, 64
L = 512
E, K = 8, 2
F = 1024
SH = 2 * F

BT = 256
BLK = 256
NB = (T * K) // BLK + E
NS = NB * BLK
NP = T * K
EPS = 1e-6
ISQ_DH = 1.0 / np.sqrt(DH)
ISQ_C = 1.0 / np.sqrt(C)

NW = 32
BW = 64
CH = NP // (NW * BW)


def _rms(x, w):
    return x * lax.rsqrt(jnp.mean(x * x, axis=-1, keepdims=True) + EPS) * w


def _proj_body(x_ref, ln1_ref, wq_ref, wkvd_ref, wku_ref, wvu_ref,
               q_ref, k_ref, v_ref):
    h = _rms(x_ref[...], ln1_ref[...])
    q_ref[...] = (jnp.dot(h, wq_ref[...], preferred_element_type=jnp.float32)
                  * ISQ_DH).astype(jnp.bfloat16)
    kvl = jnp.dot(h, wkvd_ref[...], preferred_element_type=jnp.float32)
    k_ref[...] = jnp.dot(kvl, wku_ref[...],
                         preferred_element_type=jnp.float32).astype(jnp.bfloat16)
    vv = jnp.dot(kvl, wvu_ref[...], preferred_element_type=jnp.float32)
    v_ref[...] = jnp.concatenate(
        [vv, jnp.ones((BT, 1), jnp.float32),
         jnp.zeros((BT, 128 - DH - 1), jnp.float32)],
        axis=-1).astype(jnp.bfloat16)


def _proj(x2d, ln1_w, wq, wkv_down, wk_up, wv_up, interpret=False):
    nt = T // BT
    return pl.pallas_call(
        _proj_body,
        grid=(nt,),
        in_specs=[
            pl.BlockSpec((BT, C), lambda i: (i, 0)),
            pl.BlockSpec((1, C), lambda i: (0, 0)),
            pl.BlockSpec((C, H * DH), lambda i: (0, 0)),
            pl.BlockSpec((C, L), lambda i: (0, 0)),
            pl.BlockSpec((L, DH), lambda i: (0, 0)),
            pl.BlockSpec((L, DH), lambda i: (0, 0)),
        ],
        out_specs=[
            pl.BlockSpec((BT, H * DH), lambda i: (i, 0)),
            pl.BlockSpec((BT, DH), lambda i: (i, 0)),
            pl.BlockSpec((BT, 128), lambda i: (i, 0)),
        ],
        out_shape=[
            jax.ShapeDtypeStruct((T, H * DH), jnp.bfloat16),
            jax.ShapeDtypeStruct((T, DH), jnp.bfloat16),
            jax.ShapeDtypeStruct((T, 128), jnp.bfloat16),
        ],
        interpret=interpret,
    )(x2d, ln1_w.reshape(1, C), wq, wkv_down, wk_up, wv_up)


def _attn_piece_body(i0, TK, q_ref, kt_ref, va_ref, o_ref):
    il = pl.program_id(1)
    i = i0 + il
    q = q_ref[0]
    s = jnp.dot(q, kt_ref[...], preferred_element_type=jnp.float32)
    m = jnp.max(s, axis=-1, keepdims=True)
    rr = i * BT + lax.broadcasted_iota(jnp.int32, (BT, TK), 0)
    cc = lax.broadcasted_iota(jnp.int32, (BT, TK), 1)
    p = jnp.exp(s - m) * (cc <= rr).astype(jnp.float32)
    acc = jnp.dot(p.astype(jnp.bfloat16), va_ref[...],
                  preferred_element_type=jnp.float32)
    o_ref[0] = (acc[:, :DH] * (1.0 / acc[:, DH:DH + 1])).astype(jnp.bfloat16)


def _attn_piece(qh, kt, va, i0, nq, TK, interpret=False):
    body = functools.partial(_attn_piece_body, i0, TK)
    return pl.pallas_call(
        body,
        grid=(H, nq),
        in_specs=[
            pl.BlockSpec((1, BT, DH), lambda h, i: (h, i0 + i, 0)),
            pl.BlockSpec((DH, TK), lambda h, i: (0, 0)),
            pl.BlockSpec((TK, 128), lambda h, i: (0, 0)),
        ],
        out_specs=pl.BlockSpec((1, BT, DH), lambda h, i: (h, i, 0)),
        out_shape=jax.ShapeDtypeStruct((H, nq * BT, DH), jnp.bfloat16),
        interpret=interpret,
    )(qh, kt[:, :TK], va[:TK])


def _attn(qh, kt, va, interpret=False):
    pieces = []
    for i0 in range(0, T // BT, 2):
        TK = (i0 + 2) * BT
        pieces.append(_attn_piece(qh, kt, va, i0, 2, TK, interpret))
    return jnp.concatenate(pieces, axis=1)


def _post_body(x_ref, y_ref, wo_ref, ln2_ref, rw_ref, rb_ref,
               x1_ref, h2_ref, idx_ref, wsel_ref):
    x1 = x_ref[...] + jnp.dot(y_ref[...].astype(jnp.float32), wo_ref[...],
                              preferred_element_type=jnp.float32)
    x1_ref[...] = x1
    h2 = _rms(x1, ln2_ref[...])
    h2_ref[...] = h2
    lg = jnp.dot(h2, rw_ref[...], preferred_element_type=jnp.float32) * ISQ_C
    biased = lg + rb_ref[...]
    iota_e = lax.broadcasted_iota(jnp.int32, (BT, E), 1)
    m1 = jnp.max(biased, axis=-1, keepdims=True)
    i1 = jnp.min(jnp.where(biased == m1, iota_e, E), axis=-1, keepdims=True)
    rest = jnp.where(iota_e == i1, -jnp.inf, biased)
    m2 = jnp.max(rest, axis=-1, keepdims=True)
    i2 = jnp.min(jnp.where(rest == m2, iota_e, E), axis=-1, keepdims=True)
    l1 = jnp.sum(jnp.where(iota_e == i1, lg, 0.0), axis=-1, keepdims=True)
    l2 = jnp.sum(jnp.where(iota_e == i2, lg, 0.0), axis=-1, keepdims=True)
    mx = jnp.maximum(l1, l2)
    e1 = jnp.exp(l1 - mx)
    e2 = jnp.exp(l2 - mx)
    den = e1 + e2
    idx_ref[...] = jnp.concatenate([i1, i2], axis=-1)
    wsel_ref[...] = jnp.concatenate([e1 / den, e2 / den], axis=-1)


def _post(x2d, y, wo, ln2_w, router_w, router_b, interpret=False):
    nt = T // BT
    return pl.pallas_call(
        _post_body,
        grid=(nt,),
        in_specs=[
            pl.BlockSpec((BT, C), lambda i: (i, 0)),
            pl.BlockSpec((BT, H * DH), lambda i: (i, 0)),
            pl.BlockSpec((H * DH, C), lambda i: (0, 0)),
            pl.BlockSpec((1, C), lambda i: (0, 0)),
            pl.BlockSpec((C, E), lambda i: (0, 0)),
            pl.BlockSpec((1, E), lambda i: (0, 0)),
        ],
        out_specs=[
            pl.BlockSpec((BT, C), lambda i: (i, 0)),
            pl.BlockSpec((BT, C), lambda i: (i, 0)),
            pl.BlockSpec((BT, K), lambda i: (i, 0)),
            pl.BlockSpec((BT, K), lambda i: (i, 0)),
        ],
        out_shape=[
            jax.ShapeDtypeStruct((T, C), jnp.float32),
            jax.ShapeDtypeStruct((T, C), jnp.float32),
            jax.ShapeDtypeStruct((T, K), jnp.int32),
            jax.ShapeDtypeStruct((T, K), jnp.float32),
        ],
        interpret=interpret,
    )(x2d, y, wo, ln2_w.reshape(1, C), router_w, router_b.reshape(1, E))


def _shared_body(x1_ref, h2_ref, sw1_ref, sw3_ref, sw2_ref, acc_ref):
    h2 = h2_ref[...]
    s1 = jnp.dot(h2, sw1_ref[...], preferred_element_type=jnp.float32)
    s3 = jnp.dot(h2, sw3_ref[...], preferred_element_type=jnp.float32)
    sh = jnp.dot(s1 * (s3 * jax.nn.sigmoid(s3)), sw2_ref[...],
                 preferred_element_type=jnp.float32)
    acc_ref[...] = x1_ref[...] + sh


def _shared(x1, h2, sw1, sw3, sw2, interpret=False):
    nt = T // BT
    return pl.pallas_call(
        _shared_body,
        grid=(nt,),
        in_specs=[
            pl.BlockSpec((BT, C), lambda i: (i, 0)),
            pl.BlockSpec((BT, C), lambda i: (i, 0)),
            pl.BlockSpec((C, SH), lambda i: (0, 0)),
            pl.BlockSpec((C, SH), lambda i: (0, 0)),
            pl.BlockSpec((SH, C), lambda i: (0, 0)),
        ],
        out_specs=pl.BlockSpec((BT, C), lambda i: (i, 0)),
        out_shape=jax.ShapeDtypeStruct((T, C), jnp.float32),
        interpret=interpret,
    )(x1, h2, sw1, sw3, sw2)


def _meta_body(idx_ref, slot_ref, be_ref, bv_ref):
    idx = idx_ref[...]
    il = lax.broadcasted_iota(jnp.int32, (T, 128), 1)
    oh0 = (il == idx[:, 0:1]).astype(jnp.bfloat16)
    oh1 = (il == idx[:, 1:2]).astype(jnp.bfloat16)
    ri = lax.broadcasted_iota(jnp.int32, (T, T), 0)
    ci = lax.broadcasted_iota(jnp.int32, (T, T), 1)
    ltri = (ri > ci).astype(jnp.bfloat16)
    pref0 = jnp.dot(ltri, oh0, preferred_element_type=jnp.float32)
    pref1 = jnp.dot(ltri, oh1, preferred_element_type=jnp.float32)
    oh0f = oh0.astype(jnp.float32)
    oh1f = oh1.astype(jnp.float32)
    tot0 = jnp.sum(oh0f, axis=0, keepdims=True)
    tot1 = jnp.sum(oh1f, axis=0, keepdims=True)
    pref1 = pref1 + tot0
    counts = tot0 + tot1
    nb = jnp.floor((counts + (BLK - 1)) / BLK)
    la = lax.broadcasted_iota(jnp.int32, (128, 128), 0)
    lb = lax.broadcasted_iota(jnp.int32, (128, 128), 1)
    umat = ((la <= lb) & (la < E)).astype(jnp.bfloat16)
    cum_nb = jnp.dot(nb.astype(jnp.bfloat16), umat,
                     preferred_element_type=jnp.float32)
    bstart = (cum_nb - nb) * BLK
    lane = lax.broadcasted_iota(jnp.int32, (1, 128), 1).astype(jnp.float32)
    b_col = lax.broadcasted_iota(jnp.int32, (128, 1), 0).astype(jnp.float32)
    cum_row = jnp.dot(nb.astype(jnp.bfloat16), umat,
                      preferred_element_type=jnp.float32)
    cmp_t = ((cum_row <= b_col) & (lane < E)).astype(jnp.float32)
    blk_e = jnp.sum(cmp_t, axis=1, keepdims=True)
    valid = (blk_e < E).astype(jnp.float32)
    last_e = jnp.max(jnp.where((counts > 0) & (lane < E), lane, 0.0),
                     axis=1, keepdims=True)
    blk_e_fin = jnp.where(valid > 0, jnp.minimum(blk_e, E - 1.0), last_e)
    be_ref[...] = blk_e_fin.astype(jnp.int32)
    bv_ref[...] = valid.astype(jnp.int32)
    rank0 = jnp.sum(pref0 * oh0f, axis=-1, keepdims=True)
    rank1 = jnp.sum(pref1 * oh1f, axis=-1, keepdims=True)
    base0 = jnp.sum(bstart * oh0f, axis=-1, keepdims=True)
    base1 = jnp.sum(bstart * oh1f, axis=-1, keepdims=True)
    slot0 = (rank0 + base0).astype(jnp.int32)
    slot1 = (rank1 + base1).astype(jnp.int32)
    slot_ref[...] = jnp.concatenate([slot0, slot1], axis=-1)


def _meta(idx, interpret=False):
    return pl.pallas_call(
        _meta_body,
        grid=(1,),
        in_specs=[pl.BlockSpec((T, K), lambda i: (0, 0))],
        out_specs=[
            pl.BlockSpec((T, K), lambda i: (0, 0)),
            pl.BlockSpec((128, 1), lambda i: (0, 0)),
            pl.BlockSpec((128, 1), lambda i: (0, 0)),
        ],
        out_shape=[
            jax.ShapeDtypeStruct((T, K), jnp.int32),
            jax.ShapeDtypeStruct((128, 1), jnp.int32),
            jax.ShapeDtypeStruct((128, 1), jnp.int32),
        ],
        interpret=interpret,
    )(idx)


def _sc_dispatch(h2, slots_w, tok_w):
    mesh = plsc.VectorSubcoreMesh(core_axis_name="c", subcore_axis_name="s")

    @functools.partial(
        pl.kernel,
        out_type=jax.ShapeDtypeStruct((NS, C), jnp.float32),
        mesh=mesh,
        scratch_types=[
            pltpu.VMEM((CH, BW), jnp.int32),
            pltpu.VMEM((CH, BW), jnp.int32),
            pltpu.VMEM((BW, C), jnp.float32),
            pltpu.SemaphoreType.DMA,
        ],
    )
    def k(h2_hbm, sl_hbm, tk_hbm, xs_hbm, sl_v, tk_v, rows_v, sem):
        wid = lax.axis_index("s") * 2 + lax.axis_index("c")
        pltpu.sync_copy(sl_hbm.at[wid], sl_v)
        pltpu.sync_copy(tk_hbm.at[wid], tk_v)
        for c in range(CH):
            pltpu.async_copy(h2_hbm.at[tk_v.at[c]], rows_v, sem).wait()
            pltpu.async_copy(rows_v, xs_hbm.at[sl_v.at[c]], sem).wait()

    return k(h2, slots_w, tok_w)


def _sc_combine(eout, slots_w):
    mesh = plsc.VectorSubcoreMesh(core_axis_name="c", subcore_axis_name="s")

    @functools.partial(
        pl.kernel,
        out_type=jax.ShapeDtypeStruct((NP, C), jnp.float32),
        mesh=mesh,
        scratch_types=[
            pltpu.VMEM((CH, BW), jnp.int32),
            pltpu.VMEM((BW, C), jnp.float32),
            pltpu.SemaphoreType.DMA,
        ],
    )
    def k(eo_hbm, sl_hbm, g_hbm, sl_v, rows_v, sem):
        wid = lax.axis_index("s") * 2 + lax.axis_index("c")
        pltpu.sync_copy(sl_hbm.at[wid], sl_v)
        for c in range(CH):
            pltpu.async_copy(eo_hbm.at[sl_v.at[c]], rows_v, sem).wait()
            pltpu.sync_copy(rows_v, g_hbm.at[pl.ds(wid * CH * BW + c * BW, BW)])

    return k(eout, slots_w)


def _moe_body(be_ref, bv_ref, xs_ref, w1_ref, w3_ref, w2_ref, out_ref):
    b = pl.program_id(0)

    @pl.when(bv_ref[b] != 0)
    def _():
        xs = xs_ref[...]
        t1 = jnp.dot(xs, w1_ref[0], preferred_element_type=jnp.float32)
        t3 = jnp.dot(xs, w3_ref[0], preferred_element_type=jnp.float32)
        hdn = t1 * (t3 * jax.nn.sigmoid(t3))
        out_ref[...] = jnp.dot(hdn, w2_ref[0],
                               preferred_element_type=jnp.float32)

    @pl.when(bv_ref[b] == 0)
    def _():
        out_ref[...] = jnp.zeros_like(out_ref)


def _moe(xs, ew1, ew3, ew2, block_e, block_v, interpret=False):
    grid_spec = pltpu.PrefetchScalarGridSpec(
        num_scalar_prefetch=2,
        grid=(NB,),
        in_specs=[
            pl.BlockSpec((BLK, C), lambda b, be, bv: (b, 0)),
            pl.BlockSpec((1, C, F), lambda b, be, bv: (be[b], 0, 0)),
            pl.BlockSpec((1, C, F), lambda b, be, bv: (be[b], 0, 0)),
            pl.BlockSpec((1, F, C), lambda b, be, bv: (be[b], 0, 0)),
        ],
        out_specs=pl.BlockSpec((BLK, C), lambda b, be, bv: (b, 0)),
    )
    return pl.pallas_call(
        _moe_body,
        grid_spec=grid_spec,
        out_shape=jax.ShapeDtypeStruct((NS, C), jnp.float32),
        interpret=interpret,
    )(block_e, block_v, xs, ew1, ew3, ew2)


def _fin_body(acc_ref, g0_ref, g1_ref, w_ref, o_ref):
    w = w_ref[...]
    o_ref[...] = (acc_ref[...] + w[:, 0:1] * g0_ref[...]
                  + w[:, 1:2] * g1_ref[...])


def _fin(acc, g, wsel, interpret=False):
    nt = T // BT
    return pl.pallas_call(
        _fin_body,
        grid=(nt,),
        in_specs=[
            pl.BlockSpec((BT, C), lambda i: (i, 0)),
            pl.BlockSpec((BT, C), lambda i: (i, 0)),
            pl.BlockSpec((BT, C), lambda i: (i + T // BT, 0)),
            pl.BlockSpec((BT, K), lambda i: (i, 0)),
        ],
        out_specs=pl.BlockSpec((BT, C), lambda i: (i, 0)),
        out_shape=jax.ShapeDtypeStruct((T, C), jnp.float32),
        interpret=interpret,
    )(acc, g, g, wsel)


def _forward(x, ln1_w, ln2_w, wq, wkv_down, wk_up, wv_up, wo,
             router_w, router_b, ew1, ew2, ew3, sw1, sw2, sw3,
             interpret=False):
    x2d = x.reshape(T, C)
    q, k, va = _proj(x2d, ln1_w, wq, wkv_down, wk_up, wv_up, interpret)
    qh = q.reshape(T, H, DH).transpose(1, 0, 2)
    yh = _attn(qh, k.T, va, interpret)
    y = yh.transpose(1, 0, 2).reshape(T, H * DH)
    x1, h2, idx, wsel = _post(x2d, y, wo, ln2_w, router_w, router_b, interpret)
    slots, be_c, bv_c = _meta(idx, interpret)
    block_e = be_c.reshape(128)[:NB]
    block_v = bv_c.reshape(128)[:NB]
    slots_w = slots.T.reshape(NW, CH, BW)
    tok_w = (jnp.arange(NP, dtype=jnp.int32) % T).reshape(NW, CH, BW)
    xs = _sc_dispatch(h2, slots_w, tok_w)
    acc = _shared(x1, h2, sw1, sw3, sw2, interpret)
    eout = _moe(xs, ew1, ew3, ew2, block_e, block_v, interpret)
    g = _sc_combine(eout, slots_w)
    out = _fin(acc, g, wsel, interpret)
    return out.reshape(B, T, C)


def kernel(x, ln1_w, ln2_w, wq, wkv_down, wk_up, wv_up, wo,
           router_w, router_b, ew1, ew2, ew3, sw1, sw2, sw3):
    return _forward(x, ln1_w, ln2_w, wq, wkv_down, wk_up, wv_up, wo,
                    router_w, router_b, ew1, ew2, ew3, sw1, sw2, sw3)


# speedup vs baseline: 1.0722x; 1.0722x over previous
import functools

import jax
import jax.numpy as jnp
import numpy as np
from jax import lax
from jax.experimental import pallas as pl
from jax.experimental.pallas import tpu as pltpu
from jax.experimental.pallas import tpu_sc as plsc

B, T, C = 1, 2048, 1024
H, DH = 16, 64
L = 512
E, K = 8, 2
F = 1024
SH = 2 * F

BT = 256
BLK = 256
NB = (T * K) // BLK + E
NS = NB * BLK
NP = T * K
EPS = 1e-6
ISQ_DH = 1.0 / np.sqrt(DH)
ISQ_C = 1.0 / np.sqrt(C)

NW = 32
BW = 64
CH = NP // (NW * BW)


def _rms(x, w):
    return x * lax.rsqrt(jnp.mean(x * x, axis=-1, keepdims=True) + EPS) * w


def _proj_body(x_ref, ln1_ref, wq_ref, wkvd_ref, wku_ref, wvu_ref,
               q_ref, k_ref, v_ref):
    h = _rms(x_ref[...], ln1_ref[...])
    q_ref[...] = (jnp.dot(h, wq_ref[...], preferred_element_type=jnp.float32)
                  * ISQ_DH).astype(jnp.bfloat16)
    kvl = jnp.dot(h, wkvd_ref[...], preferred_element_type=jnp.float32)
    k_ref[...] = jnp.dot(kvl, wku_ref[...],
                         preferred_element_type=jnp.float32).astype(jnp.bfloat16)
    vv = jnp.dot(kvl, wvu_ref[...], preferred_element_type=jnp.float32)
    v_ref[...] = jnp.concatenate(
        [vv, jnp.ones((BT, 1), jnp.float32),
         jnp.zeros((BT, 128 - DH - 1), jnp.float32)],
        axis=-1).astype(jnp.bfloat16)


def _proj(x2d, ln1_w, wq, wkv_down, wk_up, wv_up, interpret=False):
    nt = T // BT
    return pl.pallas_call(
        _proj_body,
        grid=(nt,),
        in_specs=[
            pl.BlockSpec((BT, C), lambda i: (i, 0)),
            pl.BlockSpec((1, C), lambda i: (0, 0)),
            pl.BlockSpec((C, H * DH), lambda i: (0, 0)),
            pl.BlockSpec((C, L), lambda i: (0, 0)),
            pl.BlockSpec((L, DH), lambda i: (0, 0)),
            pl.BlockSpec((L, DH), lambda i: (0, 0)),
        ],
        out_specs=[
            pl.BlockSpec((BT, H * DH), lambda i: (i, 0)),
            pl.BlockSpec((BT, DH), lambda i: (i, 0)),
            pl.BlockSpec((BT, 128), lambda i: (i, 0)),
        ],
        out_shape=[
            jax.ShapeDtypeStruct((T, H * DH), jnp.bfloat16),
            jax.ShapeDtypeStruct((T, DH), jnp.bfloat16),
            jax.ShapeDtypeStruct((T, 128), jnp.bfloat16),
        ],
        interpret=interpret,
    )(x2d, ln1_w.reshape(1, C), wq, wkv_down, wk_up, wv_up)


BQ = 512


def _attn_piece_body(p, TK, q_ref, kt_ref, va_ref, o_ref):
    q = q_ref[0]
    s = jnp.dot(q, kt_ref[...], preferred_element_type=jnp.float32)
    m = jnp.max(s, axis=-1, keepdims=True)
    rr = p * BQ + lax.broadcasted_iota(jnp.int32, (BQ, TK), 0)
    cc = lax.broadcasted_iota(jnp.int32, (BQ, TK), 1)
    pm = jnp.exp(s - m) * (cc <= rr).astype(jnp.float32)
    acc = jnp.dot(pm.astype(jnp.bfloat16), va_ref[...],
                  preferred_element_type=jnp.float32)
    o_ref[0] = (acc[:, :DH] * (1.0 / acc[:, DH:DH + 1])).astype(jnp.bfloat16)


def _attn_piece(qh, kt, va, p, TK, interpret=False):
    body = functools.partial(_attn_piece_body, p, TK)
    return pl.pallas_call(
        body,
        grid=(H,),
        in_specs=[
            pl.BlockSpec((1, BQ, DH), lambda h: (h, p, 0)),
            pl.BlockSpec((DH, TK), lambda h: (0, 0)),
            pl.BlockSpec((TK, 128), lambda h: (0, 0)),
        ],
        out_specs=pl.BlockSpec((1, BQ, DH), lambda h: (h, 0, 0)),
        out_shape=jax.ShapeDtypeStruct((H, BQ, DH), jnp.bfloat16),
        interpret=interpret,
    )(qh, kt[:, :TK], va[:TK])


def _attn(qh, kt, va, interpret=False):
    pieces = []
    for p in range(T // BQ):
        TK = (p + 1) * BQ
        pieces.append(_attn_piece(qh, kt, va, p, TK, interpret))
    return jnp.concatenate(pieces, axis=1)


def _post_body(x_ref, y_ref, wo_ref, ln2_ref, rw_ref, rb_ref,
               x1_ref, h2_ref, idx_ref, wsel_ref):
    x1 = x_ref[...] + jnp.dot(y_ref[...].astype(jnp.float32), wo_ref[...],
                              preferred_element_type=jnp.float32)
    x1_ref[...] = x1
    h2 = _rms(x1, ln2_ref[...])
    h2_ref[...] = h2
    lg = jnp.dot(h2, rw_ref[...], preferred_element_type=jnp.float32) * ISQ_C
    biased = lg + rb_ref[...]
    iota_e = lax.broadcasted_iota(jnp.int32, (BT, E), 1)
    m1 = jnp.max(biased, axis=-1, keepdims=True)
    i1 = jnp.min(jnp.where(biased == m1, iota_e, E), axis=-1, keepdims=True)
    rest = jnp.where(iota_e == i1, -jnp.inf, biased)
    m2 = jnp.max(rest, axis=-1, keepdims=True)
    i2 = jnp.min(jnp.where(rest == m2, iota_e, E), axis=-1, keepdims=True)
    l1 = jnp.sum(jnp.where(iota_e == i1, lg, 0.0), axis=-1, keepdims=True)
    l2 = jnp.sum(jnp.where(iota_e == i2, lg, 0.0), axis=-1, keepdims=True)
    mx = jnp.maximum(l1, l2)
    e1 = jnp.exp(l1 - mx)
    e2 = jnp.exp(l2 - mx)
    den = e1 + e2
    idx_ref[...] = jnp.concatenate([i1, i2], axis=-1)
    wsel_ref[...] = jnp.concatenate([e1 / den, e2 / den], axis=-1)


def _post(x2d, y, wo, ln2_w, router_w, router_b, interpret=False):
    nt = T // BT
    return pl.pallas_call(
        _post_body,
        grid=(nt,),
        in_specs=[
            pl.BlockSpec((BT, C), lambda i: (i, 0)),
            pl.BlockSpec((BT, H * DH), lambda i: (i, 0)),
            pl.BlockSpec((H * DH, C), lambda i: (0, 0)),
            pl.BlockSpec((1, C), lambda i: (0, 0)),
            pl.BlockSpec((C, E), lambda i: (0, 0)),
            pl.BlockSpec((1, E), lambda i: (0, 0)),
        ],
        out_specs=[
            pl.BlockSpec((BT, C), lambda i: (i, 0)),
            pl.BlockSpec((BT, C), lambda i: (i, 0)),
            pl.BlockSpec((BT, K), lambda i: (i, 0)),
            pl.BlockSpec((BT, K), lambda i: (i, 0)),
        ],
        out_shape=[
            jax.ShapeDtypeStruct((T, C), jnp.float32),
            jax.ShapeDtypeStruct((T, C), jnp.float32),
            jax.ShapeDtypeStruct((T, K), jnp.int32),
            jax.ShapeDtypeStruct((T, K), jnp.float32),
        ],
        interpret=interpret,
    )(x2d, y, wo, ln2_w.reshape(1, C), router_w, router_b.reshape(1, E))


def _shared_body(x1_ref, h2_ref, sw1_ref, sw3_ref, sw2_ref, acc_ref):
    h2 = h2_ref[...]
    s1 = jnp.dot(h2, sw1_ref[...], preferred_element_type=jnp.float32)
    s3 = jnp.dot(h2, sw3_ref[...], preferred_element_type=jnp.float32)
    sh = jnp.dot(s1 * (s3 * jax.nn.sigmoid(s3)), sw2_ref[...],
                 preferred_element_type=jnp.float32)
    acc_ref[...] = x1_ref[...] + sh


def _shared(x1, h2, sw1, sw3, sw2, interpret=False):
    nt = T // BT
    return pl.pallas_call(
        _shared_body,
        grid=(nt,),
        in_specs=[
            pl.BlockSpec((BT, C), lambda i: (i, 0)),
            pl.BlockSpec((BT, C), lambda i: (i, 0)),
            pl.BlockSpec((C, SH), lambda i: (0, 0)),
            pl.BlockSpec((C, SH), lambda i: (0, 0)),
            pl.BlockSpec((SH, C), lambda i: (0, 0)),
        ],
        out_specs=pl.BlockSpec((BT, C), lambda i: (i, 0)),
        out_shape=jax.ShapeDtypeStruct((T, C), jnp.float32),
        interpret=interpret,
    )(x1, h2, sw1, sw3, sw2)


def _meta_body(idx_ref, slot_ref, be_ref, bv_ref):
    idx = idx_ref[...]
    il = lax.broadcasted_iota(jnp.int32, (T, 128), 1)
    oh0 = (il == idx[:, 0:1]).astype(jnp.bfloat16)
    oh1 = (il == idx[:, 1:2]).astype(jnp.bfloat16)
    ri = lax.broadcasted_iota(jnp.int32, (T, T), 0)
    ci = lax.broadcasted_iota(jnp.int32, (T, T), 1)
    ltri = (ri > ci).astype(jnp.bfloat16)
    pref0 = jnp.dot(ltri, oh0, preferred_element_type=jnp.float32)
    pref1 = jnp.dot(ltri, oh1, preferred_element_type=jnp.float32)
    oh0f = oh0.astype(jnp.float32)
    oh1f = oh1.astype(jnp.float32)
    tot0 = jnp.sum(oh0f, axis=0, keepdims=True)
    tot1 = jnp.sum(oh1f, axis=0, keepdims=True)
    pref1 = pref1 + tot0
    counts = tot0 + tot1
    nb = jnp.floor((counts + (BLK - 1)) / BLK)
    la = lax.broadcasted_iota(jnp.int32, (128, 128), 0)
    lb = lax.broadcasted_iota(jnp.int32, (128, 128), 1)
    umat = ((la <= lb) & (la < E)).astype(jnp.bfloat16)
    cum_nb = jnp.dot(nb.astype(jnp.bfloat16), umat,
                     preferred_element_type=jnp.float32)
    bstart = (cum_nb - nb) * BLK
    lane = lax.broadcasted_iota(jnp.int32, (1, 128), 1).astype(jnp.float32)
    b_col = lax.broadcasted_iota(jnp.int32, (128, 1), 0).astype(jnp.float32)
    cum_row = jnp.dot(nb.astype(jnp.bfloat16), umat,
                      preferred_element_type=jnp.float32)
    cmp_t = ((cum_row <= b_col) & (lane < E)).astype(jnp.float32)
    blk_e = jnp.sum(cmp_t, axis=1, keepdims=True)
    valid = (blk_e < E).astype(jnp.float32)
    last_e = jnp.max(jnp.where((counts > 0) & (lane < E), lane, 0.0),
                     axis=1, keepdims=True)
    blk_e_fin = jnp.where(valid > 0, jnp.minimum(blk_e, E - 1.0), last_e)
    be_ref[...] = blk_e_fin.astype(jnp.int32)
    bv_ref[...] = valid.astype(jnp.int32)
    rank0 = jnp.sum(pref0 * oh0f, axis=-1, keepdims=True)
    rank1 = jnp.sum(pref1 * oh1f, axis=-1, keepdims=True)
    base0 = jnp.sum(bstart * oh0f, axis=-1, keepdims=True)
    base1 = jnp.sum(bstart * oh1f, axis=-1, keepdims=True)
    slot0 = (rank0 + base0).astype(jnp.int32)
    slot1 = (rank1 + base1).astype(jnp.int32)
    slot_ref[...] = jnp.concatenate([slot0, slot1], axis=-1)


def _meta(idx, interpret=False):
    return pl.pallas_call(
        _meta_body,
        grid=(1,),
        in_specs=[pl.BlockSpec((T, K), lambda i: (0, 0))],
        out_specs=[
            pl.BlockSpec((T, K), lambda i: (0, 0)),
            pl.BlockSpec((128, 1), lambda i: (0, 0)),
            pl.BlockSpec((128, 1), lambda i: (0, 0)),
        ],
        out_shape=[
            jax.ShapeDtypeStruct((T, K), jnp.int32),
            jax.ShapeDtypeStruct((128, 1), jnp.int32),
            jax.ShapeDtypeStruct((128, 1), jnp.int32),
        ],
        interpret=interpret,
    )(idx)


def _sc_dispatch(h2, slots_w, tok_w):
    mesh = plsc.VectorSubcoreMesh(core_axis_name="c", subcore_axis_name="s")

    @functools.partial(
        pl.kernel,
        out_type=jax.ShapeDtypeStruct((NS, C), jnp.float32),
        mesh=mesh,
        scratch_types=[
            pltpu.VMEM((CH, BW), jnp.int32),
            pltpu.VMEM((CH, BW), jnp.int32),
            pltpu.VMEM((BW, C), jnp.float32),
            pltpu.SemaphoreType.DMA,
        ],
    )
    def k(h2_hbm, sl_hbm, tk_hbm, xs_hbm, sl_v, tk_v, rows_v, sem):
        wid = lax.axis_index("s") * 2 + lax.axis_index("c")
        pltpu.sync_copy(sl_hbm.at[wid], sl_v)
        pltpu.sync_copy(tk_hbm.at[wid], tk_v)
        for c in range(CH):
            pltpu.async_copy(h2_hbm.at[tk_v.at[c]], rows_v, sem).wait()
            pltpu.async_copy(rows_v, xs_hbm.at[sl_v.at[c]], sem).wait()

    return k(h2, slots_w, tok_w)


def _sc_combine(eout, slots_w):
    mesh = plsc.VectorSubcoreMesh(core_axis_name="c", subcore_axis_name="s")

    @functools.partial(
        pl.kernel,
        out_type=jax.ShapeDtypeStruct((NP, C), jnp.float32),
        mesh=mesh,
        scratch_types=[
            pltpu.VMEM((CH, BW), jnp.int32),
            pltpu.VMEM((BW, C), jnp.float32),
            pltpu.SemaphoreType.DMA,
        ],
    )
    def k(eo_hbm, sl_hbm, g_hbm, sl_v, rows_v, sem):
        wid = lax.axis_index("s") * 2 + lax.axis_index("c")
        pltpu.sync_copy(sl_hbm.at[wid], sl_v)
        for c in range(CH):
            pltpu.async_copy(eo_hbm.at[sl_v.at[c]], rows_v, sem).wait()
            pltpu.sync_copy(rows_v, g_hbm.at[pl.ds(wid * CH * BW + c * BW, BW)])

    return k(eout, slots_w)


def _moe_body(be_ref, bv_ref, xs_ref, w1_ref, w3_ref, w2_ref, out_ref):
    b = pl.program_id(0)

    @pl.when(bv_ref[b] != 0)
    def _():
        xs = xs_ref[...]
        t1 = jnp.dot(xs, w1_ref[0], preferred_element_type=jnp.float32)
        t3 = jnp.dot(xs, w3_ref[0], preferred_element_type=jnp.float32)
        hdn = t1 * (t3 * jax.nn.sigmoid(t3))
        out_ref[...] = jnp.dot(hdn, w2_ref[0],
                               preferred_element_type=jnp.float32)

    @pl.when(bv_ref[b] == 0)
    def _():
        out_ref[...] = jnp.zeros_like(out_ref)


def _moe(xs, ew1, ew3, ew2, block_e, block_v, interpret=False):
    grid_spec = pltpu.PrefetchScalarGridSpec(
        num_scalar_prefetch=2,
        grid=(NB,),
        in_specs=[
            pl.BlockSpec((BLK, C), lambda b, be, bv: (b, 0)),
            pl.BlockSpec((1, C, F), lambda b, be, bv: (be[b], 0, 0)),
            pl.BlockSpec((1, C, F), lambda b, be, bv: (be[b], 0, 0)),
            pl.BlockSpec((1, F, C), lambda b, be, bv: (be[b], 0, 0)),
        ],
        out_specs=pl.BlockSpec((BLK, C), lambda b, be, bv: (b, 0)),
    )
    return pl.pallas_call(
        _moe_body,
        grid_spec=grid_spec,
        out_shape=jax.ShapeDtypeStruct((NS, C), jnp.float32),
        interpret=interpret,
    )(block_e, block_v, xs, ew1, ew3, ew2)


def _fin_body(acc_ref, g0_ref, g1_ref, w_ref, o_ref):
    w = w_ref[...]
    o_ref[...] = (acc_ref[...] + w[:, 0:1] * g0_ref[...]
                  + w[:, 1:2] * g1_ref[...])


def _fin(acc, g, wsel, interpret=False):
    nt = T // BT
    return pl.pallas_call(
        _fin_body,
        grid=(nt,),
        in_specs=[
            pl.BlockSpec((BT, C), lambda i: (i, 0)),
            pl.BlockSpec((BT, C), lambda i: (i, 0)),
            pl.BlockSpec((BT, C), lambda i: (i + T // BT, 0)),
            pl.BlockSpec((BT, K), lambda i: (i, 0)),
        ],
        out_specs=pl.BlockSpec((BT, C), lambda i: (i, 0)),
        out_shape=jax.ShapeDtypeStruct((T, C), jnp.float32),
        interpret=interpret,
    )(acc, g, g, wsel)


def _forward(x, ln1_w, ln2_w, wq, wkv_down, wk_up, wv_up, wo,
             router_w, router_b, ew1, ew2, ew3, sw1, sw2, sw3,
             interpret=False):
    x2d = x.reshape(T, C)
    q, k, va = _proj(x2d, ln1_w, wq, wkv_down, wk_up, wv_up, interpret)
    qh = q.reshape(T, H, DH).transpose(1, 0, 2)
    yh = _attn(qh, k.T, va, interpret)
    y = yh.transpose(1, 0, 2).reshape(T, H * DH)
    x1, h2, idx, wsel = _post(x2d, y, wo, ln2_w, router_w, router_b, interpret)
    slots, be_c, bv_c = _meta(idx, interpret)
    block_e = be_c.reshape(128)[:NB]
    block_v = bv_c.reshape(128)[:NB]
    slots_w = slots.T.reshape(NW, CH, BW)
    tok_w = (jnp.arange(NP, dtype=jnp.int32) % T).reshape(NW, CH, BW)
    xs = _sc_dispatch(h2, slots_w, tok_w)
    acc = _shared(x1, h2, sw1, sw3, sw2, interpret)
    eout = _moe(xs, ew1, ew3, ew2, block_e, block_v, interpret)
    g = _sc_combine(eout, slots_w)
    out = _fin(acc, g, wsel, interpret)
    return out.reshape(B, T, C)


def kernel(x, ln1_w, ln2_w, wq, wkv_down, wk_up, wv_up, wo,
           router_w, router_b, ew1, ew2, ew3, sw1, sw2, sw3):
    return _forward(x, ln1_w, ln2_w, wq, wkv_down, wk_up, wv_up, wo,
                    router_w, router_b, ew1, ew2, ew3, sw1, sw2, sw3)
